# baseline (speedup 1.0000x reference)
import dataclasses
import functools

import jax
import jax.numpy as jnp
from jax import lax
from jax.experimental import pallas as pl
from jax.experimental.pallas import tpu as pltpu
from jax.experimental.pallas import tpu_sc as plsc

S, DIM = 2048, 768
HEADS, HD = 12, 64
E, TOPK, HID = 8, 2, 3072
SCALE = HD ** -0.5
NA = S * TOPK
TILE = 256
NT = NA // TILE + E
NPAD = NT * TILE
NW = 32
TPW = S // NW
HALF = TPW // 2


def _ln(x, g, b):
    m = jnp.mean(x, axis=-1, keepdims=True)
    v = jnp.mean((x - m) ** 2, axis=-1, keepdims=True)
    return (x - m) * lax.rsqrt(v + 1e-5) * g + b


def _dot_t(a, b):
    return lax.dot_general(a, b, (((1,), (1,)), ((), ())),
                           preferred_element_type=jnp.float32)


def _gelu(x):
    return 0.5 * x * (1.0 + lax.erf(x * (2.0 ** -0.5)))


def _k1_body(x_ref, g1_ref, b1_ref, w_ref, o_ref):
    ln = _ln(x_ref[...], g1_ref[...], b1_ref[...])
    o_ref[...] = _dot_t(ln, w_ref[...])


def _k1(x, g1, b1, qkv_w):
    blk = 256
    return pl.pallas_call(
        _k1_body,
        grid=(S // blk,),
        in_specs=[
            pl.BlockSpec((blk, DIM), lambda i: (i, 0)),
            pl.BlockSpec((1, DIM), lambda i: (0, 0)),
            pl.BlockSpec((1, DIM), lambda i: (0, 0)),
            pl.BlockSpec((3 * DIM, DIM), lambda i: (0, 0)),
        ],
        out_specs=pl.BlockSpec((blk, 3 * DIM), lambda i: (i, 0)),
        out_shape=jax.ShapeDtypeStruct((S, 3 * DIM), jnp.float32),
    )(x, g1, b1, qkv_w)


def _k2_body(q_ref, k_ref, v_ref, o_ref):
    qq = (q_ref[...] * SCALE).astype(jnp.bfloat16)
    kk = k_ref[...].astype(jnp.bfloat16)
    vv = v_ref[...].astype(jnp.bfloat16)
    outs = []
    for sub in range(2):
        q = qq[:, sub * HD:(sub + 1) * HD]
        k = kk[:, sub * HD:(sub + 1) * HD]
        s = _dot_t(q, k).astype(jnp.bfloat16)
        m = jnp.max(s, axis=1, keepdims=True)
        p = jnp.exp(s - m)
        l = jnp.sum(p, axis=1, keepdims=True, dtype=jnp.float32)
        av = lax.dot_general(p, vv[:, sub * HD:(sub + 1) * HD],
                             (((1,), (0,)), ((), ())),
                             preferred_element_type=jnp.float32)
        outs.append(av / l)
    o_ref[...] = jnp.concatenate(outs, axis=1)


def _k2(qkv):
    hp = HEADS // 2
    col = lambda off: pl.BlockSpec((S, 2 * HD), lambda h: (0, off + h))
    return pl.pallas_call(
        _k2_body,
        grid=(hp,),
        in_specs=[col(0), col(hp), col(2 * hp)],
        out_specs=pl.BlockSpec((S, 2 * HD), lambda h: (0, h)),
        out_shape=jax.ShapeDtypeStruct((S, DIM), jnp.float32),
    )(qkv, qkv, qkv)


def _k3_body(a_ref, x_ref, pw_ref, pb_ref, g2_ref, b2_ref, gw_ref, gb_ref,
             x1_ref, ln2_ref, eid_ref, wts_ref):
    x1 = _dot_t(a_ref[...], pw_ref[...]) + pb_ref[...] + x_ref[...]
    x1_ref[...] = x1
    ln2 = _ln(x1, g2_ref[...], b2_ref[...])
    ln2_ref[...] = ln2
    logits = _dot_t(ln2, gw_ref[...]) + gb_ref[...]
    blk = logits.shape[0]
    iota = lax.broadcasted_iota(jnp.int32, (blk, E), 1)
    m1 = jnp.max(logits, axis=1, keepdims=True)
    a1 = jnp.min(jnp.where(logits == m1, iota, E), axis=1, keepdims=True)
    l2 = jnp.where(iota == a1, -jnp.inf, logits)
    m2 = jnp.max(l2, axis=1, keepdims=True)
    a2 = jnp.min(jnp.where(l2 == m2, iota, E), axis=1, keepdims=True)
    w1 = 1.0 / (1.0 + jnp.exp(m2 - m1))
    w2 = 1.0 / (1.0 + jnp.exp(m1 - m2))
    eid_ref[...] = jnp.concatenate([a1, a2], axis=1)
    wts_ref[...] = jnp.concatenate([w1, w2], axis=1)


def _k3(attn, x, proj_w, proj_b, g2, b2, gate_w, gate_b):
    blk = 256
    row = lambda d: pl.BlockSpec((blk, d), lambda i: (i, 0))
    full = lambda r, d: pl.BlockSpec((r, d), lambda i: (0, 0))
    return pl.pallas_call(
        _k3_body,
        grid=(S // blk,),
        in_specs=[
            row(DIM), row(DIM), full(DIM, DIM), full(1, DIM),
            full(1, DIM), full(1, DIM), full(E, DIM), full(1, E),
        ],
        out_specs=[row(DIM), row(DIM), pl.BlockSpec((blk, 2), lambda i: (i, 0)),
                   pl.BlockSpec((blk, 2), lambda i: (i, 0))],
        out_shape=[
            jax.ShapeDtypeStruct((S, DIM), jnp.float32),
            jax.ShapeDtypeStruct((S, DIM), jnp.float32),
            jax.ShapeDtypeStruct((S, 2), jnp.int32),
            jax.ShapeDtypeStruct((S, 2), jnp.float32),
        ],
    )(attn, x, proj_w, proj_b, g2, b2, gate_w, gate_b)


CH = 128
NCH = NA // CH


def _kr_body(e_ref, pos_ref, texp_ref, ntu_ref):
    iota8 = lax.broadcasted_iota(jnp.int32, (CH, E), 1)
    ir = lax.broadcasted_iota(jnp.int32, (CH, CH), 0)
    ic = lax.broadcasted_iota(jnp.int32, (CH, CH), 1)
    tril = (ic < ir).astype(jnp.float32)

    def pass1(i, c):
        blk = e_ref[pl.ds(i * CH, CH), :]
        oh = (blk == iota8).astype(jnp.float32)
        return c + jnp.sum(oh, axis=0, keepdims=True)

    counts = lax.fori_loop(0, NCH, pass1, jnp.zeros((1, E), jnp.float32))
    pc = jnp.floor((counts + (TILE - 1)) / TILE) * TILE
    m8 = (lax.broadcasted_iota(jnp.int32, (E, E), 0)
          < lax.broadcasted_iota(jnp.int32, (E, E), 1)).astype(jnp.float32)
    starts = lax.dot_general(pc, m8, (((1,), (0,)), ((), ())),
                             preferred_element_type=jnp.float32)

    def pass2(i, run):
        blk = e_ref[pl.ds(i * CH, CH), :]
        oh = blk == iota8
        ohf = oh.astype(jnp.float32)
        local = lax.dot_general(tril, ohf, (((1,), (0,)), ((), ())),
                                preferred_element_type=jnp.float32)
        posb = jnp.sum(jnp.where(oh, starts + local + run, 0.0),
                       axis=1, keepdims=True)
        pos_ref[pl.ds(i * CH, CH), :] = posb.astype(jnp.int32)
        return run + jnp.sum(ohf, axis=0, keepdims=True)

    lax.fori_loop(0, NCH, pass2, jnp.zeros((1, E), jnp.float32))

    jt = (lax.broadcasted_iota(jnp.int32, (NT, 1), 0) * TILE).astype(jnp.float32)
    cmp = (starts <= jt).astype(jnp.int32)
    texp_ref[...] = jnp.sum(cmp, axis=1, keepdims=True) - 1
    used = jnp.sum(pc) / TILE
    ntu_ref[...] = jnp.full((1, 1), used, jnp.float32).astype(jnp.int32)


def _k_route(e_flat):
    return pl.pallas_call(
        _kr_body,
        in_specs=[pl.BlockSpec((NA, 1), lambda: (0, 0))],
        out_specs=[pl.BlockSpec((NA, 1), lambda: (0, 0)),
                   pl.BlockSpec((NT, 1), lambda: (0, 0)),
                   pl.BlockSpec((1, 1), lambda: (0, 0))],
        out_shape=[jax.ShapeDtypeStruct((NA, 1), jnp.int32),
                   jax.ShapeDtypeStruct((NT, 1), jnp.int32),
                   jax.ShapeDtypeStruct((1, 1), jnp.int32)],
    )(e_flat)


def _k4s_body(te_ref, xs_ref, w1_ref, b1_ref, w2_ref, b2_ref, ys_ref,
              w1c_ref, w2c_ref):
    i = pl.program_id(0)

    @pl.when(i < te_ref[NT])
    def _():
        prev = te_ref[jnp.maximum(i - 1, 0)]

        @pl.when((i == 0) | (te_ref[i] != prev))
        def _():
            w1c_ref[...] = w1_ref[0].astype(jnp.bfloat16)
            w2c_ref[...] = w2_ref[0].astype(jnp.bfloat16)

        x = xs_ref[...].astype(jnp.bfloat16)
        h = _gelu(_dot_t(x, w1c_ref[...]) + b1_ref[0])
        ys_ref[...] = _dot_t(h.astype(jnp.bfloat16), w2c_ref[...]) + b2_ref[0]


def _k4_sparse(texp, xs, fc1_w, fc1_b, fc2_w, fc2_b):
    grid_spec = pltpu.PrefetchScalarGridSpec(
        num_scalar_prefetch=1,
        grid=(NT,),
        in_specs=[
            pl.BlockSpec((TILE, DIM), lambda i, te: (i, 0)),
            pl.BlockSpec((1, HID, DIM), lambda i, te: (te[i], 0, 0)),
            pl.BlockSpec((1, 1, HID), lambda i, te: (te[i], 0, 0)),
            pl.BlockSpec((1, DIM, HID), lambda i, te: (te[i], 0, 0)),
            pl.BlockSpec((1, 1, DIM), lambda i, te: (te[i], 0, 0)),
        ],
        out_specs=pl.BlockSpec((TILE, DIM), lambda i, te: (i, 0)),
        scratch_shapes=[
            pltpu.VMEM((HID, DIM), jnp.bfloat16),
            pltpu.VMEM((DIM, HID), jnp.bfloat16),
        ],
    )
    return pl.pallas_call(
        _k4s_body,
        grid_spec=grid_spec,
        out_shape=jax.ShapeDtypeStruct((NPAD, DIM), jnp.float32),
        compiler_params=pltpu.CompilerParams(
            dimension_semantics=("arbitrary",)),
    )(texp, xs, fc1_w, fc1_b, fc2_w, fc2_b)


def _dispatch_sc(ln2, pos_t):
    mesh = plsc.VectorSubcoreMesh(core_axis_name="c", subcore_axis_name="s")

    @functools.partial(
        pl.kernel, mesh=mesh,
        compiler_params=_sc_params(),
        out_type=jax.ShapeDtypeStruct((NPAD, DIM), jnp.float32),
        scratch_types=[
            pltpu.VMEM((2, HALF), jnp.int32),
            pltpu.VMEM((2, HALF), jnp.int32),
            pltpu.VMEM((2, HALF, DIM), jnp.float32),
            pltpu.SemaphoreType.DMA,
            pltpu.SemaphoreType.DMA,
        ],
    )
    def k(ln2_hbm, pos_hbm, xs_hbm, idx0_v, idx1_v, rows_v, sem0, sem1):
        wid = lax.axis_index("s") * 2 + lax.axis_index("c")
        sems = [sem0, sem1]

        def fetch(h, par):
            base = wid * TPW + h * HALF
            pltpu.sync_copy(pos_hbm.at[0, pl.ds(base, HALF)], idx0_v.at[par])
            pltpu.sync_copy(pos_hbm.at[1, pl.ds(base, HALF)], idx1_v.at[par])
            return [pltpu.async_copy(ln2_hbm.at[pl.ds(base, HALF)],
                                     rows_v.at[par], sems[par])]

        handles = fetch(0, 0)
        for h in range(2):
            par = h % 2
            for hh in handles:
                hh.wait()
            if h + 1 < 2:
                handles = fetch(h + 1, 1 - par)
            pltpu.async_copy(rows_v.at[par], xs_hbm.at[idx0_v.at[par]],
                             sems[par]).wait()
            pltpu.async_copy(rows_v.at[par], xs_hbm.at[idx1_v.at[par]],
                             sems[par]).wait()

    return k(ln2, pos_t)


def _sc_params(layout_passes=True):
    cp = pltpu.CompilerParams(use_tc_tiling_on_sc=True)
    if not layout_passes and (
            "needs_layout_passes" in pltpu.CompilerParams.__dataclass_fields__):
        cp = dataclasses.replace(cp, needs_layout_passes=False)
    return cp


QTR = TPW // 4
NQ = 4


def _combine_sc(ys, pos_t, w_t, x1):
    mesh = plsc.VectorSubcoreMesh(core_axis_name="c", subcore_axis_name="s")

    @functools.partial(
        pl.kernel, mesh=mesh,
        compiler_params=_sc_params(layout_passes=False),
        out_type=jax.ShapeDtypeStruct((S, DIM), jnp.float32),
        scratch_types=[
            pltpu.VMEM((2, QTR), jnp.int32),
            pltpu.VMEM((2, QTR), jnp.int32),
            pltpu.VMEM((2, QTR), jnp.float32),
            pltpu.VMEM((2, QTR), jnp.float32),
            pltpu.VMEM((2, QTR, DIM), jnp.float32),
            pltpu.VMEM((2, QTR, DIM), jnp.float32),
            pltpu.VMEM((2, QTR, DIM), jnp.float32),
            pltpu.SemaphoreType.DMA,
            pltpu.SemaphoreType.DMA,
        ],
    )
    def k(ys_hbm, pos_hbm, w_hbm, x1_hbm, o_hbm,
          idx0_v, idx1_v, w0_v, w1_v, y0_v, y1_v, acc_v, sem0, sem1):
        wid = lax.axis_index("s") * 2 + lax.axis_index("c")
        sems = [sem0, sem1]

        def issue(q, par):
            base = wid * TPW + q * QTR
            pltpu.sync_copy(pos_hbm.at[0, pl.ds(base, QTR)], idx0_v.at[par])
            pltpu.sync_copy(pos_hbm.at[1, pl.ds(base, QTR)], idx1_v.at[par])
            h = [pltpu.async_copy(ys_hbm.at[idx0_v.at[par]],
                                  y0_v.at[par], sems[par]),
                 pltpu.async_copy(ys_hbm.at[idx1_v.at[par]],
                                  y1_v.at[par], sems[par]),
                 pltpu.async_copy(x1_hbm.at[pl.ds(base, QTR)],
                                  acc_v.at[par], sems[par]),
                 pltpu.async_copy(w_hbm.at[0, pl.ds(base, QTR)],
                                  w0_v.at[par], sems[par]),
                 pltpu.async_copy(w_hbm.at[1, pl.ds(base, QTR)],
                                  w1_v.at[par], sems[par])]
            return h

        handles = issue(0, 0)
        for q in range(NQ):
            par = q % 2
            for h in handles:
                h.wait()
            if q + 1 < NQ:
                handles = issue(q + 1, 1 - par)

            @pl.loop(0, QTR)
            def _(j):
                lane = jnp.broadcast_to(j, (16,)).astype(jnp.int32)
                w0 = plsc.load_gather(w0_v.at[par], [lane])
                w1 = plsc.load_gather(w1_v.at[par], [lane])
                for c in range(DIM // 16):
                    sl = (j, pl.ds(c * 16, 16))
                    acc_v[par, *sl] = (acc_v[par, *sl]
                                       + w0 * y0_v[par, *sl]
                                       + w1 * y1_v[par, *sl])

            base = wid * TPW + q * QTR
            pltpu.sync_copy(acc_v.at[par], o_hbm.at[pl.ds(base, QTR)])

    return k(ys, pos_t, w_t, x1)


def kernel(x, g1, b1, qkv_w, proj_w, proj_b, g2, b2, gate_w, gate_b,
           fc1_w, fc1_b, fc2_w, fc2_b):
    xf = x.reshape(S, DIM)
    qkv = _k1(xf, g1.reshape(1, DIM), b1.reshape(1, DIM), qkv_w)
    attn = _k2(qkv)
    x1, ln2, eid, wts = _k3(attn, xf, proj_w, proj_b.reshape(1, DIM),
                            g2.reshape(1, DIM), b2.reshape(1, DIM),
                            gate_w, gate_b.reshape(1, E))
    pos, texp, ntu = _k_route(eid.reshape(NA, 1))
    pos_t = pos.reshape(S, 2).T
    w_t = wts.T
    xs = _dispatch_sc(ln2, pos_t)
    te_vec = jnp.concatenate([texp.reshape(NT), ntu.reshape(1)])
    ys = _k4_sparse(te_vec, xs, fc1_w, fc1_b.reshape(E, 1, HID),
                    fc2_w, fc2_b.reshape(E, 1, DIM))
    out = _combine_sc(ys, pos_t, w_t, x1)
    return out.reshape(x.shape)


# speedup vs baseline: 1.0085x; 1.0085x over previous
import dataclasses
import functools

import jax
import jax.numpy as jnp
from jax import lax
from jax.experimental import pallas as pl
from jax.experimental.pallas import tpu as pltpu
from jax.experimental.pallas import tpu_sc as plsc

S, DIM = 2048, 768
HEADS, HD = 12, 64
E, TOPK, HID = 8, 2, 3072
SCALE = HD ** -0.5
NA = S * TOPK
TILE = 256
NT = NA // TILE + E
NPAD = NT * TILE
NW = 32
TPW = S // NW
HALF = TPW // 2


def _ln(x, g, b):
    m = jnp.mean(x, axis=-1, keepdims=True)
    v = jnp.mean((x - m) ** 2, axis=-1, keepdims=True)
    return (x - m) * lax.rsqrt(v + 1e-5) * g + b


def _dot_t(a, b):
    return lax.dot_general(a, b, (((1,), (1,)), ((), ())),
                           preferred_element_type=jnp.float32)


def _gelu(x):
    return 0.5 * x * (1.0 + lax.erf(x * (2.0 ** -0.5)))


def _k1_body(x_ref, g1_ref, b1_ref, w_ref, o_ref):
    ln = _ln(x_ref[...], g1_ref[...], b1_ref[...])
    o_ref[...] = _dot_t(ln, w_ref[...])


def _k1(x, g1, b1, qkv_w):
    blk = 256
    return pl.pallas_call(
        _k1_body,
        grid=(S // blk,),
        in_specs=[
            pl.BlockSpec((blk, DIM), lambda i: (i, 0)),
            pl.BlockSpec((1, DIM), lambda i: (0, 0)),
            pl.BlockSpec((1, DIM), lambda i: (0, 0)),
            pl.BlockSpec((3 * DIM, DIM), lambda i: (0, 0)),
        ],
        out_specs=pl.BlockSpec((blk, 3 * DIM), lambda i: (i, 0)),
        out_shape=jax.ShapeDtypeStruct((S, 3 * DIM), jnp.float32),
    )(x, g1, b1, qkv_w)


def _k2_body(q_ref, k_ref, v_ref, o_ref):
    qq = (q_ref[...] * SCALE).astype(jnp.bfloat16)
    kk = k_ref[...].astype(jnp.bfloat16)
    vv = v_ref[...].astype(jnp.bfloat16)
    outs = []
    for sub in range(2):
        q = qq[:, sub * HD:(sub + 1) * HD]
        k = kk[:, sub * HD:(sub + 1) * HD]
        s = _dot_t(q, k).astype(jnp.bfloat16)
        m = jnp.max(s, axis=1, keepdims=True)
        p = jnp.exp(s - m)
        l = jnp.sum(p, axis=1, keepdims=True, dtype=jnp.float32)
        av = lax.dot_general(p, vv[:, sub * HD:(sub + 1) * HD],
                             (((1,), (0,)), ((), ())),
                             preferred_element_type=jnp.float32)
        outs.append(av / l)
    o_ref[...] = jnp.concatenate(outs, axis=1)


def _k2(qkv):
    hp = HEADS // 2
    col = lambda off: pl.BlockSpec((S, 2 * HD), lambda h: (0, off + h))
    return pl.pallas_call(
        _k2_body,
        grid=(hp,),
        in_specs=[col(0), col(hp), col(2 * hp)],
        out_specs=pl.BlockSpec((S, 2 * HD), lambda h: (0, h)),
        out_shape=jax.ShapeDtypeStruct((S, DIM), jnp.float32),
    )(qkv, qkv, qkv)


def _k3_body(a_ref, x_ref, pw_ref, pb_ref, g2_ref, b2_ref, gw_ref, gb_ref,
             x1_ref, ln2_ref, eid_ref, wts_ref):
    x1 = _dot_t(a_ref[...], pw_ref[...]) + pb_ref[...] + x_ref[...]
    x1_ref[...] = x1
    ln2 = _ln(x1, g2_ref[...], b2_ref[...])
    ln2_ref[...] = ln2
    logits = _dot_t(ln2, gw_ref[...]) + gb_ref[...]
    blk = logits.shape[0]
    iota = lax.broadcasted_iota(jnp.int32, (blk, E), 1)
    m1 = jnp.max(logits, axis=1, keepdims=True)
    a1 = jnp.min(jnp.where(logits == m1, iota, E), axis=1, keepdims=True)
    l2 = jnp.where(iota == a1, -jnp.inf, logits)
    m2 = jnp.max(l2, axis=1, keepdims=True)
    a2 = jnp.min(jnp.where(l2 == m2, iota, E), axis=1, keepdims=True)
    w1 = 1.0 / (1.0 + jnp.exp(m2 - m1))
    w2 = 1.0 / (1.0 + jnp.exp(m1 - m2))
    eid_ref[...] = jnp.concatenate([a1, a2], axis=1)
    wts_ref[...] = jnp.concatenate([w1, w2], axis=1)


def _k3(attn, x, proj_w, proj_b, g2, b2, gate_w, gate_b):
    blk = 256
    row = lambda d: pl.BlockSpec((blk, d), lambda i: (i, 0))
    full = lambda r, d: pl.BlockSpec((r, d), lambda i: (0, 0))
    return pl.pallas_call(
        _k3_body,
        grid=(S // blk,),
        in_specs=[
            row(DIM), row(DIM), full(DIM, DIM), full(1, DIM),
            full(1, DIM), full(1, DIM), full(E, DIM), full(1, E),
        ],
        out_specs=[row(DIM), row(DIM), pl.BlockSpec((blk, 2), lambda i: (i, 0)),
                   pl.BlockSpec((blk, 2), lambda i: (i, 0))],
        out_shape=[
            jax.ShapeDtypeStruct((S, DIM), jnp.float32),
            jax.ShapeDtypeStruct((S, DIM), jnp.float32),
            jax.ShapeDtypeStruct((S, 2), jnp.int32),
            jax.ShapeDtypeStruct((S, 2), jnp.float32),
        ],
    )(attn, x, proj_w, proj_b, g2, b2, gate_w, gate_b)


CH = 128
NCH = NA // CH


def _kr_body(e_ref, pos_ref, texp_ref, ntu_ref):
    iota8 = lax.broadcasted_iota(jnp.int32, (CH, E), 1)
    ir = lax.broadcasted_iota(jnp.int32, (CH, CH), 0)
    ic = lax.broadcasted_iota(jnp.int32, (CH, CH), 1)
    tril = (ic < ir).astype(jnp.float32)

    def pass1(i, c):
        blk = e_ref[pl.ds(i * CH, CH), :]
        oh = (blk == iota8).astype(jnp.float32)
        return c + jnp.sum(oh, axis=0, keepdims=True)

    counts = lax.fori_loop(0, NCH, pass1, jnp.zeros((1, E), jnp.float32))
    pc = jnp.floor((counts + (TILE - 1)) / TILE) * TILE
    m8 = (lax.broadcasted_iota(jnp.int32, (E, E), 0)
          < lax.broadcasted_iota(jnp.int32, (E, E), 1)).astype(jnp.float32)
    starts = lax.dot_general(pc, m8, (((1,), (0,)), ((), ())),
                             preferred_element_type=jnp.float32)

    def pass2(i, run):
        blk = e_ref[pl.ds(i * CH, CH), :]
        oh = blk == iota8
        ohf = oh.astype(jnp.float32)
        local = lax.dot_general(tril, ohf, (((1,), (0,)), ((), ())),
                                preferred_element_type=jnp.float32)
        posb = jnp.sum(jnp.where(oh, starts + local + run, 0.0),
                       axis=1, keepdims=True)
        pos_ref[pl.ds(i * CH, CH), :] = posb.astype(jnp.int32)
        return run + jnp.sum(ohf, axis=0, keepdims=True)

    lax.fori_loop(0, NCH, pass2, jnp.zeros((1, E), jnp.float32))

    jt = (lax.broadcasted_iota(jnp.int32, (NT, 1), 0) * TILE).astype(jnp.float32)
    cmp = (starts <= jt).astype(jnp.int32)
    texp_ref[...] = jnp.sum(cmp, axis=1, keepdims=True) - 1
    used = jnp.sum(pc) / TILE
    ntu_ref[...] = jnp.full((1, 1), used, jnp.float32).astype(jnp.int32)


def _k_route(e_flat):
    return pl.pallas_call(
        _kr_body,
        in_specs=[pl.BlockSpec((NA, 1), lambda: (0, 0))],
        out_specs=[pl.BlockSpec((NA, 1), lambda: (0, 0)),
                   pl.BlockSpec((NT, 1), lambda: (0, 0)),
                   pl.BlockSpec((1, 1), lambda: (0, 0))],
        out_shape=[jax.ShapeDtypeStruct((NA, 1), jnp.int32),
                   jax.ShapeDtypeStruct((NT, 1), jnp.int32),
                   jax.ShapeDtypeStruct((1, 1), jnp.int32)],
    )(e_flat)


def _k4s_body(te_ref, xs_ref, w1_ref, b1_ref, w2_ref, b2_ref, ys_ref,
              w1c_ref, w2c_ref):
    i = pl.program_id(0)

    @pl.when(i < te_ref[NT])
    def _():
        prev = te_ref[jnp.maximum(i - 1, 0)]

        @pl.when((i == 0) | (te_ref[i] != prev))
        def _():
            w1c_ref[...] = w1_ref[0].astype(jnp.bfloat16)
            w2c_ref[...] = w2_ref[0].astype(jnp.bfloat16)

        x = xs_ref[...].astype(jnp.bfloat16)
        h = _gelu(_dot_t(x, w1c_ref[...]) + b1_ref[0])
        ys_ref[...] = _dot_t(h.astype(jnp.bfloat16), w2c_ref[...]) + b2_ref[0]


def _k4_sparse(texp, xs, fc1_w, fc1_b, fc2_w, fc2_b):
    grid_spec = pltpu.PrefetchScalarGridSpec(
        num_scalar_prefetch=1,
        grid=(NT,),
        in_specs=[
            pl.BlockSpec((TILE, DIM), lambda i, te: (i, 0)),
            pl.BlockSpec((1, HID, DIM), lambda i, te: (te[i], 0, 0)),
            pl.BlockSpec((1, 1, HID), lambda i, te: (te[i], 0, 0)),
            pl.BlockSpec((1, DIM, HID), lambda i, te: (te[i], 0, 0)),
            pl.BlockSpec((1, 1, DIM), lambda i, te: (te[i], 0, 0)),
        ],
        out_specs=pl.BlockSpec((TILE, DIM), lambda i, te: (i, 0)),
        scratch_shapes=[
            pltpu.VMEM((HID, DIM), jnp.bfloat16),
            pltpu.VMEM((DIM, HID), jnp.bfloat16),
        ],
    )
    return pl.pallas_call(
        _k4s_body,
        grid_spec=grid_spec,
        out_shape=jax.ShapeDtypeStruct((NPAD, DIM), jnp.float32),
        compiler_params=pltpu.CompilerParams(
            dimension_semantics=("arbitrary",)),
    )(texp, xs, fc1_w, fc1_b, fc2_w, fc2_b)


def _dispatch_sc(ln2, pos_t):
    mesh = plsc.VectorSubcoreMesh(core_axis_name="c", subcore_axis_name="s")

    @functools.partial(
        pl.kernel, mesh=mesh,
        compiler_params=_sc_params(),
        out_type=jax.ShapeDtypeStruct((NPAD, DIM), jnp.float32),
        scratch_types=[
            pltpu.VMEM((TPW,), jnp.int32),
            pltpu.VMEM((TPW,), jnp.int32),
            pltpu.VMEM((TPW, DIM), jnp.float32),
        ],
    )
    def k(ln2_hbm, pos_hbm, xs_hbm, idx0_v, idx1_v, rows_v):
        wid = lax.axis_index("s") * 2 + lax.axis_index("c")
        base = wid * TPW
        pltpu.sync_copy(pos_hbm.at[0, pl.ds(base, TPW)], idx0_v)
        pltpu.sync_copy(pos_hbm.at[1, pl.ds(base, TPW)], idx1_v)
        pltpu.sync_copy(ln2_hbm.at[pl.ds(base, TPW)], rows_v)
        pltpu.sync_copy(rows_v, xs_hbm.at[idx0_v])
        pltpu.sync_copy(rows_v, xs_hbm.at[idx1_v])

    return k(ln2, pos_t)


def _sc_params(layout_passes=True):
    cp = pltpu.CompilerParams(use_tc_tiling_on_sc=True)
    if not layout_passes and (
            "needs_layout_passes" in pltpu.CompilerParams.__dataclass_fields__):
        cp = dataclasses.replace(cp, needs_layout_passes=False)
    return cp


QTR = TPW // 4
NQ = 4


def _combine_sc(ys, pos_t, w_t, x1):
    mesh = plsc.VectorSubcoreMesh(core_axis_name="c", subcore_axis_name="s")

    @functools.partial(
        pl.kernel, mesh=mesh,
        compiler_params=_sc_params(layout_passes=False),
        out_type=jax.ShapeDtypeStruct((S, DIM), jnp.float32),
        scratch_types=[
            pltpu.VMEM((2, QTR), jnp.int32),
            pltpu.VMEM((2, QTR), jnp.int32),
            pltpu.VMEM((2, QTR), jnp.float32),
            pltpu.VMEM((2, QTR), jnp.float32),
            pltpu.VMEM((2, QTR, DIM), jnp.float32),
            pltpu.VMEM((2, QTR, DIM), jnp.float32),
            pltpu.VMEM((2, QTR, DIM), jnp.float32),
            pltpu.SemaphoreType.DMA,
            pltpu.SemaphoreType.DMA,
        ],
    )
    def k(ys_hbm, pos_hbm, w_hbm, x1_hbm, o_hbm,
          idx0_v, idx1_v, w0_v, w1_v, y0_v, y1_v, acc_v, sem0, sem1):
        wid = lax.axis_index("s") * 2 + lax.axis_index("c")
        sems = [sem0, sem1]

        def issue(q, par):
            base = wid * TPW + q * QTR
            pltpu.sync_copy(pos_hbm.at[0, pl.ds(base, QTR)], idx0_v.at[par])
            pltpu.sync_copy(pos_hbm.at[1, pl.ds(base, QTR)], idx1_v.at[par])
            h = [pltpu.async_copy(ys_hbm.at[idx0_v.at[par]],
                                  y0_v.at[par], sems[par]),
                 pltpu.async_copy(ys_hbm.at[idx1_v.at[par]],
                                  y1_v.at[par], sems[par]),
                 pltpu.async_copy(x1_hbm.at[pl.ds(base, QTR)],
                                  acc_v.at[par], sems[par]),
                 pltpu.async_copy(w_hbm.at[0, pl.ds(base, QTR)],
                                  w0_v.at[par], sems[par]),
                 pltpu.async_copy(w_hbm.at[1, pl.ds(base, QTR)],
                                  w1_v.at[par], sems[par])]
            return h

        handles = issue(0, 0)
        for q in range(NQ):
            par = q % 2
            for h in handles:
                h.wait()
            if q + 1 < NQ:
                handles = issue(q + 1, 1 - par)

            @pl.loop(0, QTR)
            def _(j):
                lane = jnp.broadcast_to(j, (16,)).astype(jnp.int32)
                w0 = plsc.load_gather(w0_v.at[par], [lane])
                w1 = plsc.load_gather(w1_v.at[par], [lane])
                for c in range(DIM // 16):
                    sl = (j, pl.ds(c * 16, 16))
                    acc_v[par, *sl] = (acc_v[par, *sl]
                                       + w0 * y0_v[par, *sl]
                                       + w1 * y1_v[par, *sl])

            base = wid * TPW + q * QTR
            pltpu.sync_copy(acc_v.at[par], o_hbm.at[pl.ds(base, QTR)])

    return k(ys, pos_t, w_t, x1)


def kernel(x, g1, b1, qkv_w, proj_w, proj_b, g2, b2, gate_w, gate_b,
           fc1_w, fc1_b, fc2_w, fc2_b):
    xf = x.reshape(S, DIM)
    qkv = _k1(xf, g1.reshape(1, DIM), b1.reshape(1, DIM), qkv_w)
    attn = _k2(qkv)
    x1, ln2, eid, wts = _k3(attn, xf, proj_w, proj_b.reshape(1, DIM),
                            g2.reshape(1, DIM), b2.reshape(1, DIM),
                            gate_w, gate_b.reshape(1, E))
    pos, texp, ntu = _k_route(eid.reshape(NA, 1))
    pos_t = pos.reshape(S, 2).T
    w_t = wts.T
    xs = _dispatch_sc(ln2, pos_t)
    te_vec = jnp.concatenate([texp.reshape(NT), ntu.reshape(1)])
    ys = _k4_sparse(te_vec, xs, fc1_w, fc1_b.reshape(E, 1, HID),
                    fc2_w, fc2_b.reshape(E, 1, DIM))
    out = _combine_sc(ys, pos_t, w_t, x1)
    return out.reshape(x.shape)
